# baseline (speedup 1.0000x reference)
import functools

import jax
import jax.numpy as jnp
from jax import lax
from jax.experimental import pallas as pl
from jax.experimental.pallas import tpu as pltpu
from jax.experimental.pallas import tpu_sc as plsc

B = 4096
L = 200
D = 32
V = 1000000

NC = 2
NS = 16
NW = NC * NS
RW = B // NW
CH = 40
CPR = L // CH
NBUF = 4
BL = 208

CB = 81920


def _tc_body(wt_ref, tt_ref, mpk_ref):
  res = lax.dot_general(
      wt_ref[...], tt_ref[...],
      dimension_numbers=(((1,), (0,)), ((), ())),
      preferred_element_type=jnp.float32)
  lo = lax.bitcast_convert_type(
      res[0, :].astype(jnp.bfloat16), jnp.uint16).astype(jnp.int32)
  hi = lax.bitcast_convert_type(
      res[1, :].astype(jnp.bfloat16), jnp.uint16).astype(jnp.int32)
  mpk_ref[...] = lo | (hi << 16)


_tc_call = pl.pallas_call(
    _tc_body,
    grid=(pl.cdiv(V, CB),),
    in_specs=[pl.BlockSpec((2, D), lambda i: (0, 0)),
              pl.BlockSpec((D, CB), lambda i: (0, i))],
    out_specs=pl.BlockSpec((CB,), lambda i: (i,)),
    out_shape=jax.ShapeDtypeStruct((V,), jnp.int32),
    compiler_params=pltpu.CompilerParams(
        dimension_semantics=("arbitrary",),
        vmem_limit_bytes=100 * 1024 * 1024),
)

_mesh = plsc.VectorSubcoreMesh(
    core_axis_name="c", subcore_axis_name="s", num_cores=NC, num_subcores=NS)


def _prep_body(xt_hbm, out_hbm, xin_v, idxT_v):
  c = lax.axis_index("c")
  s = lax.axis_index("s")
  w = s * NC + c
  pltpu.sync_copy(xt_hbm.at[:, pl.ds(w * RW, RW)], xin_v)
  laneL = lax.iota(jnp.int32, 16) * L
  for t in range(L):
    for k in range(RW // 16):
      vals = xin_v[t, pl.ds(k * 16, 16)]
      flat = laneL + (k * 16 * L + t)
      plsc.store_scatter(idxT_v, [flat], vals)
  pltpu.sync_copy(idxT_v, out_hbm.at[pl.ds(w * (RW * L), RW * L)])


_prep_call = pl.kernel(
    _prep_body,
    out_type=jax.ShapeDtypeStruct((B * L,), jnp.int32),
    mesh=_mesh,
    scratch_types=[pltpu.VMEM((L, RW), jnp.int32),
                   pltpu.VMEM((RW * L,), jnp.int32)],
    compiler_params=pltpu.CompilerParams(
        needs_layout_passes=False, use_tc_tiling_on_sc=False),
)


def _sc_body(x_hbm, mpk_hbm, b_hbm, out_hbm, idx_v, vv, b_v, out_v, mpk_sh,
             *sems):
  c = lax.axis_index("c")
  s = lax.axis_index("s")
  w = s * NC + c

  SH = 62496
  st = pltpu.async_copy(mpk_hbm.at[pl.ds(s * SH, SH)],
                        mpk_sh.at[pl.ds(s * SH, SH)], sems[NBUF])
  idxcp = pltpu.async_copy(x_hbm.at[pl.ds(w * (RW * L), RW * L)], idx_v,
                           sems[NBUF + 1])
  pltpu.sync_copy(b_hbm, b_v)
  @pl.when(s == NS - 1)
  def _():
    pltpu.sync_copy(mpk_hbm.at[pl.ds(NS * SH, V - NS * SH)],
                    mpk_sh.at[pl.ds(NS * SH, V - NS * SH)])
  st.wait()
  idxcp.wait()
  plsc.subcore_barrier()

  def fire(buf, r):
    pltpu.async_copy(mpk_sh.at[idx_v.at[pl.ds(r * L, L)]],
                     vv.at[buf, pl.ds(0, L)], sems[buf])

  def drain(buf):
    pltpu.make_async_copy(mpk_hbm.at[pl.ds(0, L)],
                          vv.at[buf, pl.ds(0, L)], sems[buf]).wait()

  def zero_pads(buf):
    vv[buf, pl.ds(192, 16)] = jnp.zeros((16,), jnp.int32)

  lane = lax.iota(jnp.int32, 16)
  brow = b_v[pl.ds(0, 16)]
  b0 = brow[0]
  b1 = brow[1]

  def do_row(buf, r):
    drain(buf)
    z = jnp.zeros((16,), jnp.float32)
    a0, a1 = z, z
    for i in range(BL // 16):
      words = vv[buf, pl.ds(i * 16, 16)]
      p0, p1 = plsc.unpack(plsc.bitcast(words, jnp.bfloat16),
                           format=plsc.PackFormat.INTERLEAVED,
                           preferred_element_type=jnp.float32)
      a0 = a0 + p0
      a1 = a1 + p1
    o0 = jnp.sum(a0) * (1.0 / L) + b0
    o1 = jnp.sum(a1) * (1.0 / L) + b1
    plsc.store_scatter(out_v, [jnp.full((16,), r, jnp.int32), lane],
                       jnp.where(lane == 0, o0, o1), mask=lane < 2)

  for j in range(NBUF):
    zero_pads(j)
  for j in range(NBUF):
    fire(j, j)

  def group(g, _):
    for j in range(NBUF):
      r = g * NBUF + j
      do_row(j, r)
      fire(j, r + NBUF)
    return 0

  lax.fori_loop(0, RW // NBUF - 1, group, 0)
  gl = RW // NBUF - 1
  for j in range(NBUF):
    do_row(j, gl * NBUF + j)

  pltpu.sync_copy(out_v, out_hbm.at[pl.ds(w * RW, RW)])


_sc_call = pl.kernel(
    _sc_body,
    out_type=jax.ShapeDtypeStruct((B, 2), jnp.float32),
    mesh=_mesh,
    scratch_types=(
        [pltpu.VMEM((RW * L,), jnp.int32),
         pltpu.VMEM((NBUF, BL), jnp.int32),
         pltpu.VMEM((D,), jnp.float32),
         pltpu.VMEM((RW, 2), jnp.float32),
         pltpu.VMEM_SHARED((V,), jnp.int32)]
        + [pltpu.SemaphoreType.DMA] * (NBUF + 2)),
    compiler_params=pltpu.CompilerParams(
        needs_layout_passes=False, use_tc_tiling_on_sc=False),
)


@jax.jit
def kernel(x, table, W, b):
  wt = W.T.astype(jnp.float32)
  tt = table.T
  mpk = _tc_call(wt, tt)
  x_chunks = _prep_call(x.astype(jnp.int32).T)
  bpad = jnp.pad(b.astype(jnp.float32), (0, D - 2))
  return _sc_call(x_chunks, mpk, bpad)


# speedup vs baseline: 1.0126x; 1.0126x over previous
import functools

import jax
import jax.numpy as jnp
from jax import lax
from jax.experimental import pallas as pl
from jax.experimental.pallas import tpu as pltpu
from jax.experimental.pallas import tpu_sc as plsc

B = 4096
L = 200
D = 32
V = 1000000

NC = 2
NS = 16
NW = NC * NS
RW = B // NW
CH = 40
CPR = L // CH
NBUF = 4
BL = 208

CB = 65536


def _tc_body(wt_ref, tt_ref, mpk_ref):
  res = lax.dot_general(
      wt_ref[...], tt_ref[...],
      dimension_numbers=(((1,), (0,)), ((), ())),
      preferred_element_type=jnp.float32)
  lo = lax.bitcast_convert_type(
      res[0, :].astype(jnp.bfloat16), jnp.uint16).astype(jnp.int32)
  hi = lax.bitcast_convert_type(
      res[1, :].astype(jnp.bfloat16), jnp.uint16).astype(jnp.int32)
  mpk_ref[...] = lo | (hi << 16)


_tc_call = pl.pallas_call(
    _tc_body,
    grid=(pl.cdiv(V, CB),),
    in_specs=[pl.BlockSpec((2, D), lambda i: (0, 0)),
              pl.BlockSpec((D, CB), lambda i: (0, i))],
    out_specs=pl.BlockSpec((CB,), lambda i: (i,)),
    out_shape=jax.ShapeDtypeStruct((V,), jnp.int32),
    compiler_params=pltpu.CompilerParams(
        dimension_semantics=("arbitrary",),
        vmem_limit_bytes=100 * 1024 * 1024),
)

_mesh = plsc.VectorSubcoreMesh(
    core_axis_name="c", subcore_axis_name="s", num_cores=NC, num_subcores=NS)


def _prep_body(xt_hbm, out_hbm, xin_v, idxT_v):
  c = lax.axis_index("c")
  s = lax.axis_index("s")
  w = s * NC + c
  pltpu.sync_copy(xt_hbm.at[:, pl.ds(w * RW, RW)], xin_v)
  laneL = lax.iota(jnp.int32, 16) * L
  for t in range(L):
    for k in range(RW // 16):
      vals = xin_v[t, pl.ds(k * 16, 16)]
      flat = laneL + (k * 16 * L + t)
      plsc.store_scatter(idxT_v, [flat], vals)
  pltpu.sync_copy(idxT_v, out_hbm.at[pl.ds(w * (RW * L), RW * L)])


_prep_call = pl.kernel(
    _prep_body,
    out_type=jax.ShapeDtypeStruct((B * L,), jnp.int32),
    mesh=_mesh,
    scratch_types=[pltpu.VMEM((L, RW), jnp.int32),
                   pltpu.VMEM((RW * L,), jnp.int32)],
    compiler_params=pltpu.CompilerParams(
        needs_layout_passes=False, use_tc_tiling_on_sc=False),
)


def _sc_body(x_hbm, mpk_hbm, b_hbm, out_hbm, idx_v, vv, b_v, out_v, mpk_sh,
             *sems):
  c = lax.axis_index("c")
  s = lax.axis_index("s")
  w = s * NC + c

  SH = 62496
  st = pltpu.async_copy(mpk_hbm.at[pl.ds(s * SH, SH)],
                        mpk_sh.at[pl.ds(s * SH, SH)], sems[NBUF])
  idxcp = pltpu.async_copy(x_hbm.at[pl.ds(w * (RW * L), RW * L)], idx_v,
                           sems[NBUF + 1])
  pltpu.sync_copy(b_hbm, b_v)
  @pl.when(s == NS - 1)
  def _():
    pltpu.sync_copy(mpk_hbm.at[pl.ds(NS * SH, V - NS * SH)],
                    mpk_sh.at[pl.ds(NS * SH, V - NS * SH)])
  st.wait()
  idxcp.wait()
  plsc.subcore_barrier()

  def fire(buf, r):
    pltpu.async_copy(mpk_sh.at[idx_v.at[pl.ds(r * L, L)]],
                     vv.at[buf, pl.ds(0, L)], sems[buf])

  def drain(buf):
    pltpu.make_async_copy(mpk_hbm.at[pl.ds(0, L)],
                          vv.at[buf, pl.ds(0, L)], sems[buf]).wait()

  def zero_pads(buf):
    vv[buf, pl.ds(192, 16)] = jnp.zeros((16,), jnp.int32)

  lane = lax.iota(jnp.int32, 16)
  brow = b_v[pl.ds(0, 16)]
  b0 = brow[0]
  b1 = brow[1]

  def do_row(buf, r):
    drain(buf)
    z = jnp.zeros((16,), jnp.float32)
    a0, a1 = z, z
    for i in range(BL // 16):
      words = vv[buf, pl.ds(i * 16, 16)]
      p0, p1 = plsc.unpack(plsc.bitcast(words, jnp.bfloat16),
                           format=plsc.PackFormat.INTERLEAVED,
                           preferred_element_type=jnp.float32)
      a0 = a0 + p0
      a1 = a1 + p1
    o0 = jnp.sum(a0) * (1.0 / L) + b0
    o1 = jnp.sum(a1) * (1.0 / L) + b1
    plsc.store_scatter(out_v, [jnp.full((16,), r, jnp.int32), lane],
                       jnp.where(lane == 0, o0, o1), mask=lane < 2)

  for j in range(NBUF):
    zero_pads(j)
  for j in range(NBUF):
    fire(j, j)

  def group(g, _):
    for j in range(NBUF):
      r = g * NBUF + j
      do_row(j, r)
      fire(j, r + NBUF)
    return 0

  lax.fori_loop(0, RW // NBUF - 1, group, 0)
  gl = RW // NBUF - 1
  for j in range(NBUF):
    do_row(j, gl * NBUF + j)

  pltpu.sync_copy(out_v, out_hbm.at[pl.ds(w * RW, RW)])


_sc_call = pl.kernel(
    _sc_body,
    out_type=jax.ShapeDtypeStruct((B, 2), jnp.float32),
    mesh=_mesh,
    scratch_types=(
        [pltpu.VMEM((RW * L,), jnp.int32),
         pltpu.VMEM((NBUF, BL), jnp.int32),
         pltpu.VMEM((D,), jnp.float32),
         pltpu.VMEM((RW, 2), jnp.float32),
         pltpu.VMEM_SHARED((V,), jnp.int32)]
        + [pltpu.SemaphoreType.DMA] * (NBUF + 2)),
    compiler_params=pltpu.CompilerParams(
        needs_layout_passes=False, use_tc_tiling_on_sc=False),
)


@jax.jit
def kernel(x, table, W, b):
  wt = W.T.astype(jnp.float32)
  tt = table.T
  mpk = _tc_call(wt, tt)
  x_chunks = _prep_call(x.astype(jnp.int32).T)
  bpad = jnp.pad(b.astype(jnp.float32), (0, D - 2))
  return _sc_call(x_chunks, mpk, bpad)


# speedup vs baseline: 1.0135x; 1.0009x over previous
import jax
import jax.numpy as jnp
from jax import lax
from jax.experimental import pallas as pl
from jax.experimental.pallas import tpu as pltpu
from jax.experimental.pallas import tpu_sc as plsc

B = 4096
L = 200
D = 32
V = 1000000

NC = 2
NS = 16
NW = NC * NS
RW = B // NW
NBUF = 4
BL = 208

CB = 65536


def _tc_body(wt_ref, tt_ref, mpk_ref):
  res = lax.dot_general(
      wt_ref[...], tt_ref[...],
      dimension_numbers=(((1,), (0,)), ((), ())),
      preferred_element_type=jnp.float32)
  lo = lax.bitcast_convert_type(
      res[0, :].astype(jnp.bfloat16), jnp.uint16).astype(jnp.int32)
  hi = lax.bitcast_convert_type(
      res[1, :].astype(jnp.bfloat16), jnp.uint16).astype(jnp.int32)
  mpk_ref[...] = lo | (hi << 16)


_tc_call = pl.pallas_call(
    _tc_body,
    grid=(pl.cdiv(V, CB),),
    in_specs=[pl.BlockSpec((2, D), lambda i: (0, 0)),
              pl.BlockSpec((D, CB), lambda i: (0, i))],
    out_specs=pl.BlockSpec((CB,), lambda i: (i,)),
    out_shape=jax.ShapeDtypeStruct((V,), jnp.int32),
    compiler_params=pltpu.CompilerParams(
        dimension_semantics=("arbitrary",),
        vmem_limit_bytes=100 * 1024 * 1024),
)

_mesh = plsc.VectorSubcoreMesh(
    core_axis_name="c", subcore_axis_name="s", num_cores=NC, num_subcores=NS)


def _prep_body(xt_hbm, out_hbm, xin_v, idxT_v):
  c = lax.axis_index("c")
  s = lax.axis_index("s")
  w = s * NC + c
  pltpu.sync_copy(xt_hbm.at[:, pl.ds(w * RW, RW)], xin_v)
  laneL = lax.iota(jnp.int32, 16) * L
  for t in range(L):
    for k in range(RW // 16):
      vals = xin_v[t, pl.ds(k * 16, 16)]
      flat = laneL + (k * 16 * L + t)
      plsc.store_scatter(idxT_v, [flat], vals)
  pltpu.sync_copy(idxT_v, out_hbm.at[pl.ds(w * (RW * L), RW * L)])


_prep_call = pl.kernel(
    _prep_body,
    out_type=jax.ShapeDtypeStruct((B * L,), jnp.int32),
    mesh=_mesh,
    scratch_types=[pltpu.VMEM((L, RW), jnp.int32),
                   pltpu.VMEM((RW * L,), jnp.int32)],
    compiler_params=pltpu.CompilerParams(
        needs_layout_passes=False, use_tc_tiling_on_sc=False),
)


def _sc_body(x_hbm, mpk_hbm, b_hbm, out_hbm, idx_v, vv, b_v, out_v, mpk_sh,
             *sems):
  c = lax.axis_index("c")
  s = lax.axis_index("s")
  w = s * NC + c

  SH = 62496
  st = pltpu.async_copy(mpk_hbm.at[pl.ds(s * SH, SH)],
                        mpk_sh.at[pl.ds(s * SH, SH)], sems[NBUF])
  idxcp = pltpu.async_copy(x_hbm.at[pl.ds(w * (RW * L), RW * L)], idx_v,
                           sems[NBUF + 1])
  pltpu.sync_copy(b_hbm, b_v)
  @pl.when(s == NS - 1)
  def _():
    pltpu.sync_copy(mpk_hbm.at[pl.ds(NS * SH, V - NS * SH)],
                    mpk_sh.at[pl.ds(NS * SH, V - NS * SH)])
  st.wait()
  idxcp.wait()
  plsc.subcore_barrier()

  def fire(buf, r):
    pltpu.async_copy(mpk_sh.at[idx_v.at[pl.ds(r * L, L)]],
                     vv.at[buf, pl.ds(0, L)], sems[buf])

  def drain(buf):
    pltpu.make_async_copy(mpk_hbm.at[pl.ds(0, L)],
                          vv.at[buf, pl.ds(0, L)], sems[buf]).wait()

  def zero_pads(buf):
    vv[buf, pl.ds(192, 16)] = jnp.zeros((16,), jnp.int32)

  lane = lax.iota(jnp.int32, 16)
  brow = b_v[pl.ds(0, 16)]
  b0 = brow[0]
  b1 = brow[1]

  def do_row(buf, r):
    drain(buf)
    z = jnp.zeros((16,), jnp.float32)
    a0, a1 = z, z
    for i in range(BL // 16):
      words = vv[buf, pl.ds(i * 16, 16)]
      p0, p1 = plsc.unpack(plsc.bitcast(words, jnp.bfloat16),
                           format=plsc.PackFormat.INTERLEAVED,
                           preferred_element_type=jnp.float32)
      a0 = a0 + p0
      a1 = a1 + p1
    o0 = jnp.sum(a0) * (1.0 / L) + b0
    o1 = jnp.sum(a1) * (1.0 / L) + b1
    plsc.store_scatter(out_v, [jnp.full((16,), r, jnp.int32), lane],
                       jnp.where(lane == 0, o0, o1), mask=lane < 2)

  for j in range(NBUF):
    zero_pads(j)
  for j in range(NBUF):
    fire(j, j)

  def group(g, _):
    for j in range(NBUF):
      r = g * NBUF + j
      do_row(j, r)
      fire(j, r + NBUF)
    return 0

  lax.fori_loop(0, RW // NBUF - 1, group, 0)
  gl = RW // NBUF - 1
  for j in range(NBUF):
    do_row(j, gl * NBUF + j)

  pltpu.sync_copy(out_v, out_hbm.at[pl.ds(w * RW, RW)])


_sc_call = pl.kernel(
    _sc_body,
    out_type=jax.ShapeDtypeStruct((B, 2), jnp.float32),
    mesh=_mesh,
    scratch_types=(
        [pltpu.VMEM((RW * L,), jnp.int32),
         pltpu.VMEM((NBUF, BL), jnp.int32),
         pltpu.VMEM((D,), jnp.float32),
         pltpu.VMEM((RW, 2), jnp.float32),
         pltpu.VMEM_SHARED((V,), jnp.int32)]
        + [pltpu.SemaphoreType.DMA] * (NBUF + 2)),
    compiler_params=pltpu.CompilerParams(
        needs_layout_passes=False, use_tc_tiling_on_sc=False),
)


@jax.jit
def kernel(x, table, W, b):
  wt = W.T.astype(jnp.float32)
  tt = table.T
  mpk = _tc_call(wt, tt)
  x_chunks = _prep_call(x.astype(jnp.int32).T)
  bpad = jnp.pad(b.astype(jnp.float32), (0, D - 2))
  return _sc_call(x_chunks, mpk, bpad)
